# baseline (device time: 182889 ns/iter reference)
import functools
import os

import jax
import jax.numpy as jnp
from jax import lax
from jax.experimental import pallas as pl
from jax.experimental.pallas import tpu as pltpu

B, SQ, H, D = 8, 8, 16, 128
SKV_LOCAL = 1024
HALF = SKV_LOCAL // 2
HD = H * D
SCALE = D ** -0.5

_KVAR = os.environ.get("KVAR", "full")


def kernel(Q, K, V):
    def body(q_ref, k_hbm, v_hbm, o_ref,
             k_buf, v_buf, o_acc, o_recv, st_acc, st_recv,
             copy_sems, send_sems, recv_sems):
        my_x = lax.axis_index("x")
        my_y = lax.axis_index("y")
        y_peer = (my_x, 1 - my_y)
        x_peer = (1 - my_x, my_y)

        barrier = pltpu.get_barrier_semaphore()
        for peer in (y_peer, x_peer):
            pl.semaphore_signal(barrier, inc=1, device_id=peer,
                                device_id_type=pl.DeviceIdType.MESH)
        pl.semaphore_wait(barrier, 2)

        row0 = my_x * HALF

        def dma_batch(b, slot):
            cps = [
                pltpu.make_async_copy(
                    k_hbm.at[b, pl.ds(row0, HALF)],
                    k_buf.at[slot], copy_sems.at[slot, 0]),
                pltpu.make_async_copy(
                    v_hbm.at[b, pl.ds(row0, HALF)],
                    v_buf.at[slot], copy_sems.at[slot, 1]),
            ]
            for c in cps:
                c.start()
            return cps

        pending = dma_batch(0, 0)
        for b in range(B):
            slot = b % 2
            for c in pending:
                c.wait()
            if b + 1 < B:
                pending = dma_batch(b + 1, (b + 1) % 2)
            ms = []
            ls = []
            for h in range(H if _KVAR != "nocompute" else 0):
                hc = pl.ds(h * D, D)
                q_h = q_ref[b, :, hc]
                k_h = k_buf[slot, :, hc]
                v_h = v_buf[slot, :, hc]
                s = lax.dot_general(
                    q_h, k_h, (((1,), (1,)), ((), ())),
                    preferred_element_type=jnp.float32) * SCALE
                m = jnp.max(s, axis=-1)
                p = jnp.exp(s - m[:, None])
                l = jnp.sum(p, axis=-1)
                o_h = lax.dot_general(
                    p, v_h, (((1,), (0,)), ((), ())),
                    preferred_element_type=jnp.float32)
                o_acc[b, :, hc] = o_h
                ms.append(m)
                ls.append(l)
            if ms:
                st_acc[0, b] = jnp.stack(ms, axis=1)
                st_acc[1, b] = jnp.stack(ls, axis=1)

        phases = () if _KVAR == "nocomm" else (y_peer, x_peer)
        for phase, peer in enumerate(phases):
            o_rdma = pltpu.make_async_remote_copy(
                src_ref=o_acc, dst_ref=o_recv.at[phase],
                send_sem=send_sems.at[2 * phase],
                recv_sem=recv_sems.at[2 * phase],
                device_id=peer, device_id_type=pl.DeviceIdType.MESH)
            st_rdma = pltpu.make_async_remote_copy(
                src_ref=st_acc, dst_ref=st_recv.at[phase],
                send_sem=send_sems.at[2 * phase + 1],
                recv_sem=recv_sems.at[2 * phase + 1],
                device_id=peer, device_id_type=pl.DeviceIdType.MESH)
            o_rdma.start()
            st_rdma.start()
            o_rdma.wait()
            st_rdma.wait()

            m_s = st_acc[0]
            l_s = st_acc[1]
            m_p = st_recv[phase, 0]
            l_p = st_recv[phase, 1]
            m_n = jnp.maximum(m_s, m_p)
            a = jnp.exp(m_s - m_n)
            bt = jnp.exp(m_p - m_n)
            st_acc[0] = m_n
            st_acc[1] = a * l_s + bt * l_p
            a_w = jnp.repeat(a, D, axis=-1)
            bt_w = jnp.repeat(bt, D, axis=-1)
            o_acc[...] = a_w * o_acc[...] + bt_w * o_recv[phase]

        o_ref[...] = o_acc[...] / jnp.repeat(st_acc[1], D, axis=-1)

        @functools.partial(pl.run_scoped, sem=pltpu.SemaphoreType.REGULAR)
        def _(sem):
            for peer in (y_peer, x_peer):
                pl.semaphore_signal(sem, inc=1, device_id=peer,
                                    device_id_type=pl.DeviceIdType.MESH)
            pl.semaphore_wait(sem, 2)

    out_flat = pl.pallas_call(
        body,
        out_shape=jax.ShapeDtypeStruct((B, SQ, HD), jnp.float32),
        in_specs=[
            pl.BlockSpec(memory_space=pltpu.VMEM),
            pl.BlockSpec(memory_space=pl.ANY),
            pl.BlockSpec(memory_space=pl.ANY),
        ],
        out_specs=pl.BlockSpec(memory_space=pltpu.VMEM),
        scratch_shapes=[
            pltpu.VMEM((2, HALF, HD), jnp.float32),
            pltpu.VMEM((2, HALF, HD), jnp.float32),
            pltpu.VMEM((B, SQ, HD), jnp.float32),
            pltpu.VMEM((2, B, SQ, HD), jnp.float32),
            pltpu.VMEM((2, B, SQ, H), jnp.float32),
            pltpu.VMEM((2, 2, B, SQ, H), jnp.float32),
            pltpu.SemaphoreType.DMA((2, 2)),
            pltpu.SemaphoreType.DMA((4,)),
            pltpu.SemaphoreType.DMA((4,)),
        ],
        compiler_params=pltpu.CompilerParams(
            collective_id=0, vmem_limit_bytes=64 * 1024 * 1024),
    )(Q.reshape(B, SQ, HD), K.reshape(B, SKV_LOCAL, HD),
      V.reshape(B, SKV_LOCAL, HD))
    return out_flat.reshape(B, SQ, H, D)


# device time: 78863 ns/iter; 2.3191x vs baseline; 2.3191x over previous
import functools
import os

import jax
import jax.numpy as jnp
from jax import lax
from jax.experimental import pallas as pl
from jax.experimental.pallas import tpu as pltpu

B, SQ, H, D = 8, 8, 16, 128
SKV_LOCAL = 1024
HALF = SKV_LOCAL // 2
G = 2
NG = H // G
SCALE = D ** -0.5

_KVAR = os.environ.get("KVAR", "full")


def kernel(Q, K, V):
    def body(q_ref, k_hbm, v_hbm, o_ref,
             k_buf, v_buf, o_acc, o_recv, st_acc, st_recv,
             copy_sems, send_sems, recv_sems):
        my_x = lax.axis_index("x")
        my_y = lax.axis_index("y")
        y_peer = (my_x, 1 - my_y)
        x_peer = (1 - my_x, my_y)

        barrier = pltpu.get_barrier_semaphore()
        for peer in (y_peer, x_peer):
            pl.semaphore_signal(barrier, inc=1, device_id=peer,
                                device_id_type=pl.DeviceIdType.MESH)
        pl.semaphore_wait(barrier, 2)

        row0 = my_x * HALF

        def dma_batch(b, slot):
            cps = []
            for j in range(NG):
                cps.append(pltpu.make_async_copy(
                    k_hbm.at[b, pl.ds(row0, HALF), pl.ds(G * j, G)],
                    k_buf.at[slot, j], copy_sems.at[slot, 0, j]))
                cps.append(pltpu.make_async_copy(
                    v_hbm.at[b, pl.ds(row0, HALF), pl.ds(G * j, G)],
                    v_buf.at[slot, j], copy_sems.at[slot, 1, j]))
            for c in cps:
                c.start()
            return cps

        pending = dma_batch(0, 0)
        for b in range(B):
            slot = b % 2
            for c in pending:
                c.wait()
            if b + 1 < B:
                pending = dma_batch(b + 1, (b + 1) % 2)
            ms = []
            ls = []
            for h in range(H if _KVAR != "nocompute" else 0):
                j, i = divmod(h, G)
                q_h = q_ref[b, :, h, :]
                k_h = k_buf[slot, j, :, i, :]
                v_h = v_buf[slot, j, :, i, :]
                s = lax.dot_general(
                    q_h, k_h, (((1,), (1,)), ((), ())),
                    preferred_element_type=jnp.float32) * SCALE
                m = jnp.max(s, axis=-1)
                p = jnp.exp(s - m[:, None])
                l = jnp.sum(p, axis=-1)
                o_h = lax.dot_general(
                    p, v_h, (((1,), (0,)), ((), ())),
                    preferred_element_type=jnp.float32)
                o_acc[b, :, h, :] = o_h
                ms.append(m)
                ls.append(l)
            if ms:
                st_acc[0, b] = jnp.stack(ms, axis=1)
                st_acc[1, b] = jnp.stack(ls, axis=1)

        phases = () if _KVAR == "nocomm" else (y_peer, x_peer)
        for phase, peer in enumerate(phases):
            o_rdma = pltpu.make_async_remote_copy(
                src_ref=o_acc, dst_ref=o_recv.at[phase],
                send_sem=send_sems.at[2 * phase],
                recv_sem=recv_sems.at[2 * phase],
                device_id=peer, device_id_type=pl.DeviceIdType.MESH)
            st_rdma = pltpu.make_async_remote_copy(
                src_ref=st_acc, dst_ref=st_recv.at[phase],
                send_sem=send_sems.at[2 * phase + 1],
                recv_sem=recv_sems.at[2 * phase + 1],
                device_id=peer, device_id_type=pl.DeviceIdType.MESH)
            o_rdma.start()
            st_rdma.start()
            o_rdma.wait()
            st_rdma.wait()

            m_s = st_acc[0]
            l_s = st_acc[1]
            m_p = st_recv[phase, 0]
            l_p = st_recv[phase, 1]
            m_n = jnp.maximum(m_s, m_p)
            a = jnp.exp(m_s - m_n)
            bt = jnp.exp(m_p - m_n)
            st_acc[0] = m_n
            st_acc[1] = a * l_s + bt * l_p
            o_acc[...] = (a[..., None] * o_acc[...]
                          + bt[..., None] * o_recv[phase])

        o_ref[...] = o_acc[...] / st_acc[1][..., None]

        @functools.partial(pl.run_scoped, sem=pltpu.SemaphoreType.REGULAR)
        def _(sem):
            for peer in (y_peer, x_peer):
                pl.semaphore_signal(sem, inc=1, device_id=peer,
                                    device_id_type=pl.DeviceIdType.MESH)
            pl.semaphore_wait(sem, 2)

    return pl.pallas_call(
        body,
        out_shape=jax.ShapeDtypeStruct((B, SQ, H, D), jnp.float32),
        in_specs=[
            pl.BlockSpec(memory_space=pltpu.VMEM),
            pl.BlockSpec(memory_space=pl.ANY),
            pl.BlockSpec(memory_space=pl.ANY),
        ],
        out_specs=pl.BlockSpec(memory_space=pltpu.VMEM),
        scratch_shapes=[
            pltpu.VMEM((2, NG, HALF, G, D), jnp.float32),
            pltpu.VMEM((2, NG, HALF, G, D), jnp.float32),
            pltpu.VMEM((B, SQ, H, D), jnp.float32),
            pltpu.VMEM((2, B, SQ, H, D), jnp.float32),
            pltpu.VMEM((2, B, SQ, H), jnp.float32),
            pltpu.VMEM((2, 2, B, SQ, H), jnp.float32),
            pltpu.SemaphoreType.DMA((2, 2, NG)),
            pltpu.SemaphoreType.DMA((4,)),
            pltpu.SemaphoreType.DMA((4,)),
        ],
        compiler_params=pltpu.CompilerParams(
            collective_id=0, vmem_limit_bytes=64 * 1024 * 1024),
    )(Q, K, V)


# device time: 46248 ns/iter; 3.9545x vs baseline; 1.7052x over previous
import functools
import os

import jax
import jax.numpy as jnp
from jax import lax
from jax.experimental import pallas as pl
from jax.experimental.pallas import tpu as pltpu

B, SQ, H, D = 8, 8, 16, 128
SKV_LOCAL = 1024
HALF = SKV_LOCAL // 2
NSLOT = 3
SCALE = D ** -0.5

_KVAR = os.environ.get("KVAR", "full")


def kernel(Q, K, V):
    def body(q_ref, k_hbm, v_hbm, o_ref,
             k_buf, v_buf, o_acc, o_recv, st_acc, st_recv,
             copy_sems, send_sems, recv_sems):
        my_x = lax.axis_index("x")
        my_y = lax.axis_index("y")
        peers = ((my_x, 1 - my_y), (1 - my_x, my_y))

        barrier = pltpu.get_barrier_semaphore()
        for peer in peers:
            pl.semaphore_signal(barrier, inc=1, device_id=peer,
                                device_id_type=pl.DeviceIdType.MESH)
        pl.semaphore_wait(barrier, 2)

        row0 = my_x * HALF

        def dma_batch(b, slot):
            cps = []
            for h in range(H):
                cps.append(pltpu.make_async_copy(
                    k_hbm.at[b, pl.ds(row0, HALF), h],
                    k_buf.at[slot, h], copy_sems.at[slot, 0, h]))
                cps.append(pltpu.make_async_copy(
                    v_hbm.at[b, pl.ds(row0, HALF), h],
                    v_buf.at[slot, h], copy_sems.at[slot, 1, h]))
            for c in cps:
                c.start()
            return cps

        def exchange(phase, b):
            o_rdma = pltpu.make_async_remote_copy(
                src_ref=o_acc.at[b], dst_ref=o_recv.at[phase, b],
                send_sem=send_sems.at[phase, b, 0],
                recv_sem=recv_sems.at[phase, b, 0],
                device_id=peers[phase], device_id_type=pl.DeviceIdType.MESH)
            st_rdma = pltpu.make_async_remote_copy(
                src_ref=st_acc.at[b], dst_ref=st_recv.at[phase, b],
                send_sem=send_sems.at[phase, b, 1],
                recv_sem=recv_sems.at[phase, b, 1],
                device_id=peers[phase], device_id_type=pl.DeviceIdType.MESH)
            o_rdma.start()
            st_rdma.start()
            return (o_rdma, st_rdma)

        def combine(phase, b, rdmas):
            for r in rdmas:
                r.wait()
            m_s = st_acc[b, 0]
            l_s = st_acc[b, 1]
            m_p = st_recv[phase, b, 0]
            l_p = st_recv[phase, b, 1]
            m_n = jnp.maximum(m_s, m_p)
            a = jnp.exp(m_s - m_n)
            bt = jnp.exp(m_p - m_n)
            st_acc[b, 0] = m_n
            st_acc[b, 1] = a * l_s + bt * l_p
            o_acc[b] = a[..., None] * o_acc[b] + bt[..., None] * o_recv[phase, b]

        def finish(b):
            o_ref[b] = o_acc[b] / st_acc[b, 1][..., None]

        do_comm = _KVAR != "nocomm"
        inflight = {}

        pending = dma_batch(0, 0)
        for b in range(B):
            for c in pending:
                c.wait()
            if b + 1 < B:
                pending = dma_batch(b + 1, (b + 1) % NSLOT)
            slot = b % NSLOT
            ms = []
            ls = []
            for h in range(H if _KVAR != "nocompute" else 0):
                q_h = q_ref[b, :, h, :]
                k_h = k_buf[slot, h]
                v_h = v_buf[slot, h]
                s = lax.dot_general(
                    q_h, k_h, (((1,), (1,)), ((), ())),
                    preferred_element_type=jnp.float32) * SCALE
                m = jnp.max(s, axis=-1)
                p = jnp.exp(s - m[:, None])
                l = jnp.sum(p, axis=-1)
                o_h = lax.dot_general(
                    p, v_h, (((1,), (0,)), ((), ())),
                    preferred_element_type=jnp.float32)
                o_acc[b, :, h, :] = o_h
                ms.append(m)
                ls.append(l)
            if ms:
                st_acc[b, 0] = jnp.stack(ms, axis=1)
                st_acc[b, 1] = jnp.stack(ls, axis=1)

            if do_comm:
                inflight[(0, b)] = exchange(0, b)
                if b >= 1:
                    combine(0, b - 1, inflight.pop((0, b - 1)))
                    inflight[(1, b - 1)] = exchange(1, b - 1)
                if b >= 2:
                    combine(1, b - 2, inflight.pop((1, b - 2)))
                    finish(b - 2)

        if do_comm:
            combine(0, B - 1, inflight.pop((0, B - 1)))
            inflight[(1, B - 1)] = exchange(1, B - 1)
            combine(1, B - 2, inflight.pop((1, B - 2)))
            finish(B - 2)
            combine(1, B - 1, inflight.pop((1, B - 1)))
            finish(B - 1)
        else:
            for b in range(B):
                finish(b)

        @functools.partial(pl.run_scoped, sem=pltpu.SemaphoreType.REGULAR)
        def _(sem):
            for peer in peers:
                pl.semaphore_signal(sem, inc=1, device_id=peer,
                                    device_id_type=pl.DeviceIdType.MESH)
            pl.semaphore_wait(sem, 2)

    return pl.pallas_call(
        body,
        out_shape=jax.ShapeDtypeStruct((B, SQ, H, D), jnp.float32),
        in_specs=[
            pl.BlockSpec(memory_space=pltpu.VMEM),
            pl.BlockSpec(memory_space=pl.ANY),
            pl.BlockSpec(memory_space=pl.ANY),
        ],
        out_specs=pl.BlockSpec(memory_space=pltpu.VMEM),
        scratch_shapes=[
            pltpu.VMEM((NSLOT, H, HALF, D), jnp.float32),
            pltpu.VMEM((NSLOT, H, HALF, D), jnp.float32),
            pltpu.VMEM((B, SQ, H, D), jnp.float32),
            pltpu.VMEM((2, B, SQ, H, D), jnp.float32),
            pltpu.VMEM((B, 2, SQ, H), jnp.float32),
            pltpu.VMEM((2, B, 2, SQ, H), jnp.float32),
            pltpu.SemaphoreType.DMA((NSLOT, 2, H)),
            pltpu.SemaphoreType.DMA((2, B, 2)),
            pltpu.SemaphoreType.DMA((2, B, 2)),
        ],
        compiler_params=pltpu.CompilerParams(
            collective_id=0, vmem_limit_bytes=64 * 1024 * 1024),
    )(Q, K, V)
